# initial kernel scaffold (unmeasured)
import jax
import jax.numpy as jnp
from jax import lax
from jax.experimental import pallas as pl
from jax.experimental.pallas import tpu as pltpu

N_DEV = 4
M = 2048
D = 2048
EPS = 1e-6


def kernel(partial, gamma):
    x = partial.reshape(N_DEV * M, D)
    g = gamma.reshape(1, D)

    def body(x_ref, g_ref, out_ref, recv_bufs, sbuf, stage,
             copy_sem, out_sem, send_sems, recv_sems):
        p = lax.axis_index("i")

        def mod(a):
            return lax.rem(a + 2 * N_DEV, N_DEV)

        left = mod(p - 1)
        right = mod(p + 1)

        barrier = pltpu.get_barrier_semaphore()
        for nbr in (left, right):
            pl.semaphore_signal(barrier, inc=1, device_id=(nbr,),
                                device_id_type=pl.DeviceIdType.MESH)
        pl.semaphore_wait(barrier, 2)

        def load_chunk(c):
            cp = pltpu.make_async_copy(
                x_ref.at[pl.ds(c * M, M), :], stage, copy_sem)
            cp.start()
            cp.wait()

        load_chunk(mod(p - 1))
        sbuf[...] = stage[...].astype(jnp.bfloat16)
        rdma = pltpu.make_async_remote_copy(
            src_ref=sbuf, dst_ref=recv_bufs.at[0],
            send_sem=send_sems.at[0], recv_sem=recv_sems.at[0],
            device_id=(right,), device_id_type=pl.DeviceIdType.MESH)
        rdma.start()
        load_chunk(mod(p - 2))
        rdma.wait()

        for s in range(1, N_DEV - 1):
            recv_bufs[s - 1, ...] = (
                recv_bufs[s - 1, ...].astype(jnp.float32) + stage[...]
            ).astype(jnp.bfloat16)
            rdma = pltpu.make_async_remote_copy(
                src_ref=recv_bufs.at[s - 1], dst_ref=recv_bufs.at[s],
                send_sem=send_sems.at[s], recv_sem=recv_sems.at[s],
                device_id=(right,), device_id_type=pl.DeviceIdType.MESH)
            rdma.start()
            load_chunk(mod(p - s - 2))
            rdma.wait()

        stage[...] = recv_bufs[N_DEV - 2, ...].astype(jnp.float32) + stage[...]
        ms = jnp.mean(stage[...] * stage[...], axis=-1, keepdims=True)
        stage[...] = stage[...] * lax.rsqrt(ms + EPS) * g_ref[...]
        cp = pltpu.make_async_copy(stage, out_ref, out_sem)
        cp.start()
        cp.wait()

    return pl.pallas_call(
        body,
        out_shape=jax.ShapeDtypeStruct((M, D), jnp.float32),
        in_specs=[
            pl.BlockSpec(memory_space=pltpu.ANY),
            pl.BlockSpec(memory_space=pltpu.VMEM),
        ],
        out_specs=pl.BlockSpec(memory_space=pltpu.ANY),
        scratch_shapes=[
            pltpu.VMEM((N_DEV - 1, M, D), jnp.bfloat16),
            pltpu.VMEM((M, D), jnp.bfloat16),
            pltpu.VMEM((M, D), jnp.float32),
            pltpu.SemaphoreType.DMA,
            pltpu.SemaphoreType.DMA,
            pltpu.SemaphoreType.DMA((N_DEV - 1,)),
            pltpu.SemaphoreType.DMA((N_DEV - 1,)),
        ],
        compiler_params=pltpu.CompilerParams(collective_id=0),
    )(x, g)


# baseline (device time: 309915 ns/iter reference)
import jax
import jax.numpy as jnp
from jax import lax
from jax.experimental import pallas as pl
from jax.experimental.pallas import tpu as pltpu

N_DEV = 4
M = 2048
D = 2048
EPS = 1e-6


def kernel(partial, gamma):
    x = partial.reshape(N_DEV * M, D)
    g = gamma.reshape(1, D)

    def body(x_ref, g_ref, out_ref, recv_bufs, sbuf, stage,
             copy_sem, out_sem, send_sems, recv_sems):
        p = lax.axis_index("i")

        def mod(a):
            return lax.rem(a + 2 * N_DEV, N_DEV)

        left = mod(p - 1)
        right = mod(p + 1)

        barrier = pltpu.get_barrier_semaphore()
        for nbr in (left, right):
            pl.semaphore_signal(barrier, inc=1, device_id=(nbr,),
                                device_id_type=pl.DeviceIdType.MESH)
        pl.semaphore_wait(barrier, 2)

        def load_chunk(c):
            cp = pltpu.make_async_copy(
                x_ref.at[pl.ds(c * M, M), :], stage, copy_sem)
            cp.start()
            cp.wait()

        TILE = 256
        n_tiles = M // TILE

        def for_tiles(fn):
            def loop_body(t, carry):
                fn(pl.ds(t * TILE, TILE))
                return carry
            lax.fori_loop(0, n_tiles, loop_body, 0)

        load_chunk(mod(p - 1))

        def cast_tile(sl):
            sbuf[sl, :] = stage[sl, :].astype(jnp.bfloat16)

        for_tiles(cast_tile)
        rdma = pltpu.make_async_remote_copy(
            src_ref=sbuf, dst_ref=recv_bufs.at[0],
            send_sem=send_sems.at[0], recv_sem=recv_sems.at[0],
            device_id=(right,), device_id_type=pl.DeviceIdType.MESH)
        rdma.start()
        load_chunk(mod(p - 2))
        rdma.wait()

        for s in range(1, N_DEV - 1):
            def acc_tile(sl, s=s):
                recv_bufs[s - 1, sl, :] = (
                    recv_bufs[s - 1, sl, :].astype(jnp.float32) + stage[sl, :]
                ).astype(jnp.bfloat16)

            for_tiles(acc_tile)
            rdma = pltpu.make_async_remote_copy(
                src_ref=recv_bufs.at[s - 1], dst_ref=recv_bufs.at[s],
                send_sem=send_sems.at[s], recv_sem=recv_sems.at[s],
                device_id=(right,), device_id_type=pl.DeviceIdType.MESH)
            rdma.start()
            load_chunk(mod(p - s - 2))
            rdma.wait()

        def norm_tile(sl):
            y = recv_bufs[N_DEV - 2, sl, :].astype(jnp.float32) + stage[sl, :]
            ms = jnp.mean(y * y, axis=-1, keepdims=True)
            stage[sl, :] = y * lax.rsqrt(ms + EPS) * g_ref[...]

        for_tiles(norm_tile)
        cp = pltpu.make_async_copy(stage, out_ref, out_sem)
        cp.start()
        cp.wait()

    return pl.pallas_call(
        body,
        out_shape=jax.ShapeDtypeStruct((M, D), jnp.float32),
        in_specs=[
            pl.BlockSpec(memory_space=pltpu.MemorySpace.HBM),
            pl.BlockSpec(memory_space=pltpu.MemorySpace.VMEM),
        ],
        out_specs=pl.BlockSpec(memory_space=pltpu.MemorySpace.HBM),
        scratch_shapes=[
            pltpu.MemorySpace.VMEM((N_DEV - 1, M, D), jnp.bfloat16),
            pltpu.MemorySpace.VMEM((M, D), jnp.bfloat16),
            pltpu.MemorySpace.VMEM((M, D), jnp.float32),
            pltpu.SemaphoreType.DMA,
            pltpu.SemaphoreType.DMA,
            pltpu.SemaphoreType.DMA((N_DEV - 1,)),
            pltpu.SemaphoreType.DMA((N_DEV - 1,)),
        ],
        compiler_params=pltpu.CompilerParams(
            collective_id=0,
            vmem_limit_bytes=60 * 1024 * 1024,
        ),
    )(x, g)


# device time: 176599 ns/iter; 1.7549x vs baseline; 1.7549x over previous
import jax
import jax.numpy as jnp
from jax import lax
from jax.experimental import pallas as pl
from jax.experimental.pallas import tpu as pltpu

N_DEV = 4
M = 2048
H = M // 2
D = 2048
EPS = 1e-6


def kernel(partial, gamma):
    x = partial.reshape(N_DEV * M, D)
    g = gamma.reshape(1, D)

    def body(x_ref, g_ref, out_ref,
             recv_r, recv_l, sbuf_r, sbuf_l, stage_r, stage_l,
             copy_sem_r, copy_sem_l, out_sems,
             send_sems_r, recv_sems_r, send_sems_l, recv_sems_l):
        p = lax.axis_index("i")

        def mod(a):
            return lax.rem(a + 2 * N_DEV, N_DEV)

        left = mod(p - 1)
        right = mod(p + 1)

        barrier = pltpu.get_barrier_semaphore()
        for nbr in (left, right):
            pl.semaphore_signal(barrier, inc=1, device_id=(nbr,),
                                device_id_type=pl.DeviceIdType.MESH)
        pl.semaphore_wait(barrier, 2)

        def load_chunks(c_r, c_l):
            cp_r = pltpu.make_async_copy(
                x_ref.at[pl.ds(c_r * M, H), :], stage_r, copy_sem_r)
            cp_l = pltpu.make_async_copy(
                x_ref.at[pl.ds(c_l * M + H, H), :], stage_l, copy_sem_l)
            cp_r.start()
            cp_l.start()
            cp_r.wait()
            cp_l.wait()

        TILE = 256
        n_tiles = H // TILE

        def for_tiles(fn):
            def loop_body(t, carry):
                fn(pl.ds(t * TILE, TILE))
                return carry
            lax.fori_loop(0, n_tiles, loop_body, 0)

        def start_pair(src_r, src_l, s):
            rd_r = pltpu.make_async_remote_copy(
                src_ref=src_r, dst_ref=recv_r.at[s],
                send_sem=send_sems_r.at[s], recv_sem=recv_sems_r.at[s],
                device_id=(right,), device_id_type=pl.DeviceIdType.MESH)
            rd_l = pltpu.make_async_remote_copy(
                src_ref=src_l, dst_ref=recv_l.at[s],
                send_sem=send_sems_l.at[s], recv_sem=recv_sems_l.at[s],
                device_id=(left,), device_id_type=pl.DeviceIdType.MESH)
            rd_r.start()
            rd_l.start()
            return rd_r, rd_l

        load_chunks(mod(p - 1), mod(p + 1))

        def cast_tile(sl):
            sbuf_r[sl, :] = stage_r[sl, :].astype(jnp.bfloat16)
            sbuf_l[sl, :] = stage_l[sl, :].astype(jnp.bfloat16)

        for_tiles(cast_tile)
        rd_r, rd_l = start_pair(sbuf_r, sbuf_l, 0)
        load_chunks(mod(p - 2), mod(p + 2))
        rd_r.wait()
        rd_l.wait()

        for s in range(1, N_DEV - 1):
            def acc_tile(sl, s=s):
                recv_r[s - 1, sl, :] = (
                    recv_r[s - 1, sl, :].astype(jnp.float32) + stage_r[sl, :]
                ).astype(jnp.bfloat16)
                recv_l[s - 1, sl, :] = (
                    recv_l[s - 1, sl, :].astype(jnp.float32) + stage_l[sl, :]
                ).astype(jnp.bfloat16)

            for_tiles(acc_tile)
            rd_r, rd_l = start_pair(recv_r.at[s - 1], recv_l.at[s - 1], s)
            load_chunks(mod(p - s - 2), mod(p + s + 2))
            rd_r.wait()
            rd_l.wait()

        def norm_tile(sl):
            y = recv_r[N_DEV - 2, sl, :].astype(jnp.float32) + stage_r[sl, :]
            ms = jnp.mean(y * y, axis=-1, keepdims=True)
            stage_r[sl, :] = y * lax.rsqrt(ms + EPS) * g_ref[...]
            z = recv_l[N_DEV - 2, sl, :].astype(jnp.float32) + stage_l[sl, :]
            mz = jnp.mean(z * z, axis=-1, keepdims=True)
            stage_l[sl, :] = z * lax.rsqrt(mz + EPS) * g_ref[...]

        for_tiles(norm_tile)
        cp_r = pltpu.make_async_copy(
            stage_r, out_ref.at[pl.ds(0, H), :], out_sems.at[0])
        cp_l = pltpu.make_async_copy(
            stage_l, out_ref.at[pl.ds(H, H), :], out_sems.at[1])
        cp_r.start()
        cp_l.start()
        cp_r.wait()
        cp_l.wait()

    return pl.pallas_call(
        body,
        out_shape=jax.ShapeDtypeStruct((M, D), jnp.float32),
        in_specs=[
            pl.BlockSpec(memory_space=pltpu.MemorySpace.HBM),
            pl.BlockSpec(memory_space=pltpu.MemorySpace.VMEM),
        ],
        out_specs=pl.BlockSpec(memory_space=pltpu.MemorySpace.HBM),
        scratch_shapes=[
            pltpu.MemorySpace.VMEM((N_DEV - 1, H, D), jnp.bfloat16),
            pltpu.MemorySpace.VMEM((N_DEV - 1, H, D), jnp.bfloat16),
            pltpu.MemorySpace.VMEM((H, D), jnp.bfloat16),
            pltpu.MemorySpace.VMEM((H, D), jnp.bfloat16),
            pltpu.MemorySpace.VMEM((H, D), jnp.float32),
            pltpu.MemorySpace.VMEM((H, D), jnp.float32),
            pltpu.SemaphoreType.DMA,
            pltpu.SemaphoreType.DMA,
            pltpu.SemaphoreType.DMA((2,)),
            pltpu.SemaphoreType.DMA((N_DEV - 1,)),
            pltpu.SemaphoreType.DMA((N_DEV - 1,)),
            pltpu.SemaphoreType.DMA((N_DEV - 1,)),
            pltpu.SemaphoreType.DMA((N_DEV - 1,)),
        ],
        compiler_params=pltpu.CompilerParams(
            collective_id=0,
            vmem_limit_bytes=60 * 1024 * 1024,
        ),
    )(x, g)


# device time: 171697 ns/iter; 1.8050x vs baseline; 1.0286x over previous
import jax
import jax.numpy as jnp
from jax import lax
from jax.experimental import pallas as pl
from jax.experimental.pallas import tpu as pltpu

N_DEV = 4
M = 2048
H = M // 2
D = 2048
EPS = 1e-6


def kernel(partial, gamma):
    x = partial.reshape(N_DEV * M, D)
    g = gamma.reshape(1, D)

    def body(x_ref, g_ref, out_ref,
             recv_r, recv_l, sbuf_r, sbuf_l, cbuf_r, cbuf_l,
             stage_r, stage_l,
             copy_sem_r, copy_sem_l, out_sems,
             send_sems_r, recv_sems_r, send_sems_l, recv_sems_l):
        p = lax.axis_index("i")

        def mod(a):
            return lax.rem(a + 2 * N_DEV, N_DEV)

        left = mod(p - 1)
        right = mod(p + 1)

        barrier = pltpu.get_barrier_semaphore()
        for nbr in (left, right):
            pl.semaphore_signal(barrier, inc=1, device_id=(nbr,),
                                device_id_type=pl.DeviceIdType.MESH)
        pl.semaphore_wait(barrier, 2)

        def load_chunks(c_r, c_l):
            cp_r = pltpu.make_async_copy(
                x_ref.at[pl.ds(c_r * M, H), :], stage_r, copy_sem_r)
            cp_l = pltpu.make_async_copy(
                x_ref.at[pl.ds(c_l * M + H, H), :], stage_l, copy_sem_l)
            cp_r.start()
            cp_l.start()
            cp_r.wait()
            cp_l.wait()

        TILE = 256
        n_tiles = H // TILE

        def for_tiles(fn):
            def loop_body(t, carry):
                fn(pl.ds(t * TILE, TILE))
                return carry
            lax.fori_loop(0, n_tiles, loop_body, 0)

        def start_pair(src_r, src_l, s):
            rd_r = pltpu.make_async_remote_copy(
                src_ref=src_r, dst_ref=recv_r.at[s],
                send_sem=send_sems_r.at[s], recv_sem=recv_sems_r.at[s],
                device_id=(right,), device_id_type=pl.DeviceIdType.MESH)
            rd_l = pltpu.make_async_remote_copy(
                src_ref=src_l, dst_ref=recv_l.at[s],
                send_sem=send_sems_l.at[s], recv_sem=recv_sems_l.at[s],
                device_id=(left,), device_id_type=pl.DeviceIdType.MESH)
            rd_r.start()
            rd_l.start()
            return rd_r, rd_l

        load_chunks(mod(p - 1), mod(p + 1))

        def cast_tile(sl):
            sbuf_r[sl, :] = stage_r[sl, :].astype(jnp.bfloat16)
            sbuf_l[sl, :] = stage_l[sl, :].astype(jnp.bfloat16)

        for_tiles(cast_tile)
        rd_r, rd_l = start_pair(sbuf_r, sbuf_l, 0)
        load_chunks(mod(p - 2), mod(p + 2))

        def precast_tile(sl):
            cbuf_r[sl, :] = stage_r[sl, :].astype(jnp.bfloat16)
            cbuf_l[sl, :] = stage_l[sl, :].astype(jnp.bfloat16)

        for_tiles(precast_tile)
        rd_r.wait()
        rd_l.wait()

        for s in range(1, N_DEV - 1):
            def acc_tile(sl, s=s):
                recv_r[s - 1, sl, :] = recv_r[s - 1, sl, :] + cbuf_r[sl, :]
                recv_l[s - 1, sl, :] = recv_l[s - 1, sl, :] + cbuf_l[sl, :]

            for_tiles(acc_tile)
            rd_r, rd_l = start_pair(recv_r.at[s - 1], recv_l.at[s - 1], s)
            load_chunks(mod(p - s - 2), mod(p + s + 2))
            if s < N_DEV - 2:
                for_tiles(precast_tile)
            rd_r.wait()
            rd_l.wait()

        def norm_and_store(t, carry):
            sl = pl.ds(t * TILE, TILE)
            y = recv_r[N_DEV - 2, sl, :].astype(jnp.float32) + stage_r[sl, :]
            ms = jnp.mean(y * y, axis=-1, keepdims=True)
            stage_r[sl, :] = y * lax.rsqrt(ms + EPS) * g_ref[...]
            pltpu.make_async_copy(
                stage_r.at[sl, :], out_ref.at[pl.ds(t * TILE, TILE), :],
                out_sems.at[0]).start()
            z = recv_l[N_DEV - 2, sl, :].astype(jnp.float32) + stage_l[sl, :]
            mz = jnp.mean(z * z, axis=-1, keepdims=True)
            stage_l[sl, :] = z * lax.rsqrt(mz + EPS) * g_ref[...]
            pltpu.make_async_copy(
                stage_l.at[sl, :], out_ref.at[pl.ds(H + t * TILE, TILE), :],
                out_sems.at[1]).start()
            return carry

        lax.fori_loop(0, n_tiles, norm_and_store, 0)

        def drain(t, carry):
            sl = pl.ds(0, TILE)
            pltpu.make_async_copy(
                stage_r.at[sl, :], out_ref.at[sl, :], out_sems.at[0]).wait()
            pltpu.make_async_copy(
                stage_l.at[sl, :], out_ref.at[sl, :], out_sems.at[1]).wait()
            return carry

        lax.fori_loop(0, n_tiles, drain, 0)

    return pl.pallas_call(
        body,
        out_shape=jax.ShapeDtypeStruct((M, D), jnp.float32),
        in_specs=[
            pl.BlockSpec(memory_space=pltpu.MemorySpace.HBM),
            pl.BlockSpec(memory_space=pltpu.MemorySpace.VMEM),
        ],
        out_specs=pl.BlockSpec(memory_space=pltpu.MemorySpace.HBM),
        scratch_shapes=[
            pltpu.MemorySpace.VMEM((N_DEV - 1, H, D), jnp.bfloat16),
            pltpu.MemorySpace.VMEM((N_DEV - 1, H, D), jnp.bfloat16),
            pltpu.MemorySpace.VMEM((H, D), jnp.bfloat16),
            pltpu.MemorySpace.VMEM((H, D), jnp.bfloat16),
            pltpu.MemorySpace.VMEM((H, D), jnp.bfloat16),
            pltpu.MemorySpace.VMEM((H, D), jnp.bfloat16),
            pltpu.MemorySpace.VMEM((H, D), jnp.float32),
            pltpu.MemorySpace.VMEM((H, D), jnp.float32),
            pltpu.SemaphoreType.DMA,
            pltpu.SemaphoreType.DMA,
            pltpu.SemaphoreType.DMA((2,)),
            pltpu.SemaphoreType.DMA((N_DEV - 1,)),
            pltpu.SemaphoreType.DMA((N_DEV - 1,)),
            pltpu.SemaphoreType.DMA((N_DEV - 1,)),
            pltpu.SemaphoreType.DMA((N_DEV - 1,)),
        ],
        compiler_params=pltpu.CompilerParams(
            collective_id=0,
            vmem_limit_bytes=63 * 1024 * 1024,
        ),
    )(x, g)


# device time: 163812 ns/iter; 1.8919x vs baseline; 1.0481x over previous
import jax
import jax.numpy as jnp
from jax import lax
from jax.experimental import pallas as pl
from jax.experimental.pallas import tpu as pltpu

N_DEV = 4
M = 2048
H = M // 2
S = 2
HS = H // S
D = 2048
TILE = 256
EPS = 1e-6


def kernel(partial, gamma):
    x = partial.reshape(N_DEV * M, D)
    g = gamma.reshape(1, D)

    def body(x_ref, g_ref, out_ref,
             recv_r, recv_l, sbuf_r, sbuf_l, cbuf_r, cbuf_l,
             stage_r, stage_l,
             copy_sem_r, copy_sem_l, out_sems,
             send_sems_r, recv_sems_r, send_sems_l, recv_sems_l):
        p = lax.axis_index("i")

        def mod(a):
            return lax.rem(a + 2 * N_DEV, N_DEV)

        left = mod(p - 1)
        right = mod(p + 1)

        barrier = pltpu.get_barrier_semaphore()
        for nbr in (left, right):
            pl.semaphore_signal(barrier, inc=1, device_id=(nbr,),
                                device_id_type=pl.DeviceIdType.MESH)
        pl.semaphore_wait(barrier, 2)

        def load_chunks(c_r, c_l):
            cp_r = pltpu.make_async_copy(
                x_ref.at[pl.ds(c_r * M, H), :], stage_r, copy_sem_r)
            cp_l = pltpu.make_async_copy(
                x_ref.at[pl.ds(c_l * M + H, H), :], stage_l, copy_sem_l)
            cp_r.start()
            cp_l.start()
            cp_r.wait()
            cp_l.wait()

        def tiled(base, nrows, fn):
            def loop_body(t, carry):
                fn(pl.ds(base + t * TILE, TILE))
                return carry
            lax.fori_loop(0, nrows // TILE, loop_body, 0)

        def cast_stream(k, dst_r, dst_l):
            def f(sl):
                dst_r[sl, :] = stage_r[sl, :].astype(jnp.bfloat16)
                dst_l[sl, :] = stage_l[sl, :].astype(jnp.bfloat16)
            tiled(k * HS, HS, f)

        def precast_full():
            def f(sl):
                cbuf_r[sl, :] = stage_r[sl, :].astype(jnp.bfloat16)
                cbuf_l[sl, :] = stage_l[sl, :].astype(jnp.bfloat16)
            tiled(0, H, f)

        def acc_stream(k, s):
            def f(sl):
                recv_r[s - 1, sl, :] = recv_r[s - 1, sl, :] + cbuf_r[sl, :]
                recv_l[s - 1, sl, :] = recv_l[s - 1, sl, :] + cbuf_l[sl, :]
            tiled(k * HS, HS, f)

        def start_pair(s, k, src_r, src_l):
            row = pl.ds(k * HS, HS)
            rd_r = pltpu.make_async_remote_copy(
                src_ref=src_r, dst_ref=recv_r.at[s, row, :],
                send_sem=send_sems_r.at[s, k], recv_sem=recv_sems_r.at[s, k],
                device_id=(right,), device_id_type=pl.DeviceIdType.MESH)
            rd_l = pltpu.make_async_remote_copy(
                src_ref=src_l, dst_ref=recv_l.at[s, row, :],
                send_sem=send_sems_l.at[s, k], recv_sem=recv_sems_l.at[s, k],
                device_id=(left,), device_id_type=pl.DeviceIdType.MESH)
            rd_r.start()
            rd_l.start()
            return rd_r, rd_l

        rds = {}

        load_chunks(mod(p - 1), mod(p + 1))
        for k in range(S):
            cast_stream(k, sbuf_r, sbuf_l)
            row = pl.ds(k * HS, HS)
            rds[0, k] = start_pair(0, k, sbuf_r.at[row, :], sbuf_l.at[row, :])
        load_chunks(mod(p - 2), mod(p + 2))
        precast_full()

        for s in range(1, N_DEV - 1):
            for k in range(S):
                rd_r, rd_l = rds[s - 1, k]
                rd_r.wait_recv()
                rd_l.wait_recv()
                acc_stream(k, s)
                row = pl.ds(k * HS, HS)
                rds[s, k] = start_pair(
                    s, k, recv_r.at[s - 1, row, :], recv_l.at[s - 1, row, :])
            load_chunks(mod(p - s - 2), mod(p + s + 2))
            if s < N_DEV - 2:
                precast_full()

        for k in range(S):
            rd_r, rd_l = rds[N_DEV - 2, k]
            rd_r.wait_recv()
            rd_l.wait_recv()

            def norm_store(t, carry, k=k):
                sl = pl.ds(k * HS + t * TILE, TILE)
                y = (recv_r[N_DEV - 2, sl, :].astype(jnp.float32)
                     + stage_r[sl, :])
                ms = jnp.mean(y * y, axis=-1, keepdims=True)
                stage_r[sl, :] = y * lax.rsqrt(ms + EPS) * g_ref[...]
                pltpu.make_async_copy(
                    stage_r.at[sl, :], out_ref.at[sl, :],
                    out_sems.at[0]).start()
                z = (recv_l[N_DEV - 2, sl, :].astype(jnp.float32)
                     + stage_l[sl, :])
                mz = jnp.mean(z * z, axis=-1, keepdims=True)
                stage_l[sl, :] = z * lax.rsqrt(mz + EPS) * g_ref[...]
                pltpu.make_async_copy(
                    stage_l.at[sl, :],
                    out_ref.at[pl.ds(H + k * HS + t * TILE, TILE), :],
                    out_sems.at[1]).start()
                return carry

            lax.fori_loop(0, HS // TILE, norm_store, 0)

        for key in rds:
            rd_r, rd_l = rds[key]
            rd_r.wait_send()
            rd_l.wait_send()

        def drain(t, carry):
            sl = pl.ds(0, TILE)
            pltpu.make_async_copy(
                stage_r.at[sl, :], out_ref.at[sl, :], out_sems.at[0]).wait()
            pltpu.make_async_copy(
                stage_l.at[sl, :], out_ref.at[sl, :], out_sems.at[1]).wait()
            return carry

        lax.fori_loop(0, H // TILE, drain, 0)

    return pl.pallas_call(
        body,
        out_shape=jax.ShapeDtypeStruct((M, D), jnp.float32),
        in_specs=[
            pl.BlockSpec(memory_space=pltpu.MemorySpace.HBM),
            pl.BlockSpec(memory_space=pltpu.MemorySpace.VMEM),
        ],
        out_specs=pl.BlockSpec(memory_space=pltpu.MemorySpace.HBM),
        scratch_shapes=[
            pltpu.MemorySpace.VMEM((N_DEV - 1, H, D), jnp.bfloat16),
            pltpu.MemorySpace.VMEM((N_DEV - 1, H, D), jnp.bfloat16),
            pltpu.MemorySpace.VMEM((H, D), jnp.bfloat16),
            pltpu.MemorySpace.VMEM((H, D), jnp.bfloat16),
            pltpu.MemorySpace.VMEM((H, D), jnp.bfloat16),
            pltpu.MemorySpace.VMEM((H, D), jnp.bfloat16),
            pltpu.MemorySpace.VMEM((H, D), jnp.float32),
            pltpu.MemorySpace.VMEM((H, D), jnp.float32),
            pltpu.SemaphoreType.DMA,
            pltpu.SemaphoreType.DMA,
            pltpu.SemaphoreType.DMA((2,)),
            pltpu.SemaphoreType.DMA((N_DEV - 1, S)),
            pltpu.SemaphoreType.DMA((N_DEV - 1, S)),
            pltpu.SemaphoreType.DMA((N_DEV - 1, S)),
            pltpu.SemaphoreType.DMA((N_DEV - 1, S)),
        ],
        compiler_params=pltpu.CompilerParams(
            collective_id=0,
            vmem_limit_bytes=63 * 1024 * 1024,
        ),
    )(x, g)


# device time: 162409 ns/iter; 1.9082x vs baseline; 1.0086x over previous
import jax
import jax.numpy as jnp
from jax import lax
from jax.experimental import pallas as pl
from jax.experimental.pallas import tpu as pltpu

N_DEV = 4
M = 2048
H = M // 2
S = 4
HS = H // S
D = 2048
TILE = 256
EPS = 1e-6


def kernel(partial, gamma):
    x = partial.reshape(N_DEV * M, D)
    g = gamma.reshape(1, D)

    def body(x_ref, g_ref, out_ref,
             recv_r, recv_l, sbuf_r, sbuf_l, cbuf_r, cbuf_l,
             stage_r, stage_l,
             copy_sem_r, copy_sem_l, out_sems,
             send_sems_r, recv_sems_r, send_sems_l, recv_sems_l):
        p = lax.axis_index("i")

        def mod(a):
            return lax.rem(a + 2 * N_DEV, N_DEV)

        left = mod(p - 1)
        right = mod(p + 1)

        barrier = pltpu.get_barrier_semaphore()
        for nbr in (left, right):
            pl.semaphore_signal(barrier, inc=1, device_id=(nbr,),
                                device_id_type=pl.DeviceIdType.MESH)
        pl.semaphore_wait(barrier, 2)

        def load_chunks(c_r, c_l):
            cp_r = pltpu.make_async_copy(
                x_ref.at[pl.ds(c_r * M, H), :], stage_r, copy_sem_r)
            cp_l = pltpu.make_async_copy(
                x_ref.at[pl.ds(c_l * M + H, H), :], stage_l, copy_sem_l)
            cp_r.start()
            cp_l.start()
            cp_r.wait()
            cp_l.wait()

        def tiled(base, nrows, fn):
            def loop_body(t, carry):
                fn(pl.ds(base + t * TILE, TILE))
                return carry
            lax.fori_loop(0, nrows // TILE, loop_body, 0)

        def cast_stream(k, dst_r, dst_l):
            def f(sl):
                dst_r[sl, :] = stage_r[sl, :].astype(jnp.bfloat16)
                dst_l[sl, :] = stage_l[sl, :].astype(jnp.bfloat16)
            tiled(k * HS, HS, f)

        def precast_full():
            def f(sl):
                cbuf_r[sl, :] = stage_r[sl, :].astype(jnp.bfloat16)
                cbuf_l[sl, :] = stage_l[sl, :].astype(jnp.bfloat16)
            tiled(0, H, f)

        def acc_stream(k, s):
            def f(sl):
                recv_r[s - 1, sl, :] = recv_r[s - 1, sl, :] + cbuf_r[sl, :]
                recv_l[s - 1, sl, :] = recv_l[s - 1, sl, :] + cbuf_l[sl, :]
            tiled(k * HS, HS, f)

        def start_pair(s, k, src_r, src_l):
            row = pl.ds(k * HS, HS)
            rd_r = pltpu.make_async_remote_copy(
                src_ref=src_r, dst_ref=recv_r.at[s, row, :],
                send_sem=send_sems_r.at[s, k], recv_sem=recv_sems_r.at[s, k],
                device_id=(right,), device_id_type=pl.DeviceIdType.MESH)
            rd_l = pltpu.make_async_remote_copy(
                src_ref=src_l, dst_ref=recv_l.at[s, row, :],
                send_sem=send_sems_l.at[s, k], recv_sem=recv_sems_l.at[s, k],
                device_id=(left,), device_id_type=pl.DeviceIdType.MESH)
            rd_r.start()
            rd_l.start()
            return rd_r, rd_l

        rds = {}

        load_chunks(mod(p - 1), mod(p + 1))
        for k in range(S):
            cast_stream(k, sbuf_r, sbuf_l)
            row = pl.ds(k * HS, HS)
            rds[0, k] = start_pair(0, k, sbuf_r.at[row, :], sbuf_l.at[row, :])
        load_chunks(mod(p - 2), mod(p + 2))
        precast_full()

        for s in range(1, N_DEV - 1):
            for k in range(S):
                rd_r, rd_l = rds[s - 1, k]
                rd_r.wait_recv()
                rd_l.wait_recv()
                acc_stream(k, s)
                row = pl.ds(k * HS, HS)
                rds[s, k] = start_pair(
                    s, k, recv_r.at[s - 1, row, :], recv_l.at[s - 1, row, :])
            load_chunks(mod(p - s - 2), mod(p + s + 2))
            if s < N_DEV - 2:
                precast_full()

        for k in range(S):
            rd_r, rd_l = rds[N_DEV - 2, k]
            rd_r.wait_recv()
            rd_l.wait_recv()

            def norm_store(t, carry, k=k):
                sl = pl.ds(k * HS + t * TILE, TILE)
                y = (recv_r[N_DEV - 2, sl, :].astype(jnp.float32)
                     + stage_r[sl, :])
                ms = jnp.mean(y * y, axis=-1, keepdims=True)
                stage_r[sl, :] = y * lax.rsqrt(ms + EPS) * g_ref[...]
                pltpu.make_async_copy(
                    stage_r.at[sl, :], out_ref.at[sl, :],
                    out_sems.at[0]).start()
                z = (recv_l[N_DEV - 2, sl, :].astype(jnp.float32)
                     + stage_l[sl, :])
                mz = jnp.mean(z * z, axis=-1, keepdims=True)
                stage_l[sl, :] = z * lax.rsqrt(mz + EPS) * g_ref[...]
                pltpu.make_async_copy(
                    stage_l.at[sl, :],
                    out_ref.at[pl.ds(H + k * HS + t * TILE, TILE), :],
                    out_sems.at[1]).start()
                return carry

            lax.fori_loop(0, HS // TILE, norm_store, 0)

        for key in rds:
            rd_r, rd_l = rds[key]
            rd_r.wait_send()
            rd_l.wait_send()

        def drain(t, carry):
            sl = pl.ds(0, TILE)
            pltpu.make_async_copy(
                stage_r.at[sl, :], out_ref.at[sl, :], out_sems.at[0]).wait()
            pltpu.make_async_copy(
                stage_l.at[sl, :], out_ref.at[sl, :], out_sems.at[1]).wait()
            return carry

        lax.fori_loop(0, H // TILE, drain, 0)

    return pl.pallas_call(
        body,
        out_shape=jax.ShapeDtypeStruct((M, D), jnp.float32),
        in_specs=[
            pl.BlockSpec(memory_space=pltpu.MemorySpace.HBM),
            pl.BlockSpec(memory_space=pltpu.MemorySpace.VMEM),
        ],
        out_specs=pl.BlockSpec(memory_space=pltpu.MemorySpace.HBM),
        scratch_shapes=[
            pltpu.MemorySpace.VMEM((N_DEV - 1, H, D), jnp.bfloat16),
            pltpu.MemorySpace.VMEM((N_DEV - 1, H, D), jnp.bfloat16),
            pltpu.MemorySpace.VMEM((H, D), jnp.bfloat16),
            pltpu.MemorySpace.VMEM((H, D), jnp.bfloat16),
            pltpu.MemorySpace.VMEM((H, D), jnp.bfloat16),
            pltpu.MemorySpace.VMEM((H, D), jnp.bfloat16),
            pltpu.MemorySpace.VMEM((H, D), jnp.float32),
            pltpu.MemorySpace.VMEM((H, D), jnp.float32),
            pltpu.SemaphoreType.DMA,
            pltpu.SemaphoreType.DMA,
            pltpu.SemaphoreType.DMA((2,)),
            pltpu.SemaphoreType.DMA((N_DEV - 1, S)),
            pltpu.SemaphoreType.DMA((N_DEV - 1, S)),
            pltpu.SemaphoreType.DMA((N_DEV - 1, S)),
            pltpu.SemaphoreType.DMA((N_DEV - 1, S)),
        ],
        compiler_params=pltpu.CompilerParams(
            collective_id=0,
            vmem_limit_bytes=63 * 1024 * 1024,
        ),
    )(x, g)
